# baseline (speedup 1.0000x reference)
import functools

import jax
import jax.numpy as jnp
from jax.experimental import pallas as pl
from jax.experimental.pallas import tpu as pltpu

_RING = 3
_SUB = 512


def _fused_gcn_kernel(x_ref, w1_ref, w2_ref, b1_ref, b2_ref, adj_hbm,
                      out_ref, adjT_ref, ring_ref, s1t_ref, s2t_ref,
                      copy_sems, *, tm, num_classes, nr):
    phase = pl.program_id(0)
    i = pl.program_id(1)
    row0 = pl.multiple_of(i * tm, tm)

    n_sub = nr * (tm // _SUB)

    def dma(j):
        return pltpu.make_async_copy(
            adj_hbm.at[pl.ds(j * _SUB, _SUB), :],
            ring_ref.at[jax.lax.rem(j, _RING)],
            copy_sems.at[j],
        )

    @pl.when(phase == 0)
    def _():
        @pl.when(i == 0)
        def _():
            for j in range(_RING):
                dma(j).start()

        p = jnp.dot(x_ref[...], w1_ref[...],
                    preferred_element_type=jnp.float32)
        s1t_ref[:, pl.ds(row0, tm)] = p.astype(jnp.bfloat16).T

    @pl.when(phase == 1)
    def _():
        for off in range(tm // _SUB):
            j = i * (tm // _SUB) + off
            dma(j).wait()
            adjT_ref[:, pl.ds(j * _SUB, _SUB)] = (
                ring_ref[jax.lax.rem(j, _RING)].T)

            @pl.when(j + _RING < n_sub)
            def _():
                dma(j + _RING).start()

        ht = jnp.dot(s1t_ref[...], adjT_ref[:, pl.ds(row0, tm)],
                     preferred_element_type=jnp.float32)
        ht = jnp.maximum(ht + b1_ref[...].T, 0.0)
        s2t_ref[:, pl.ds(row0, tm)] = jax.lax.dot_general(
            w2_ref[...], ht.astype(jnp.bfloat16),
            (((0,), (0,)), ((), ())),
            preferred_element_type=jnp.float32
        ).astype(s2t_ref.dtype)

    @pl.when(phase == 2)
    def _():
        zt = jnp.dot(s2t_ref[...], adjT_ref[:, pl.ds(row0, tm)],
                     preferred_element_type=jnp.float32)
        z = zt.T + b2_ref[...]
        lane = jax.lax.broadcasted_iota(jnp.int32, z.shape, 1)
        z = jnp.where(lane < num_classes, z, jnp.float32(-1e30))
        m = jnp.max(z, axis=1, keepdims=True)
        shifted = z - m
        lse = jnp.log(jnp.sum(jnp.exp(shifted), axis=1, keepdims=True))
        out_ref[...] = (shifted - lse).astype(out_ref.dtype)


@functools.partial(jax.jit, static_argnames=("n_nodes", "num_classes", "tm"))
def _gcn_forward(x_p, adj_p, w1_p, b1_p, w2_p, b2_p, *, n_nodes, num_classes,
                 tm):
    N_pad, F_pad = x_p.shape
    H_pad = w1_p.shape[1]
    C_pad = w2_p.shape[1]
    nr = N_pad // tm

    out_p = pl.pallas_call(
        functools.partial(_fused_gcn_kernel, tm=tm, num_classes=num_classes,
                          nr=nr),
        out_shape=jax.ShapeDtypeStruct((N_pad, C_pad), jnp.float32),
        grid=(3, nr),
        in_specs=[
            pl.BlockSpec((tm, F_pad),
                         lambda p, i: (jnp.where(p == 0, i, nr - 1), 0)),
            pl.BlockSpec((F_pad, H_pad), lambda p, i: (0, 0)),
            pl.BlockSpec((H_pad, C_pad), lambda p, i: (0, 0)),
            pl.BlockSpec((1, H_pad), lambda p, i: (0, 0)),
            pl.BlockSpec((1, C_pad), lambda p, i: (0, 0)),
            pl.BlockSpec(memory_space=pl.ANY),
        ],
        out_specs=pl.BlockSpec((tm, C_pad),
                               lambda p, i: (jnp.where(p == 2, i, 0), 0)),
        scratch_shapes=[
            pltpu.VMEM((N_pad, N_pad), jnp.bfloat16),
            pltpu.VMEM((_RING, _SUB, N_pad), jnp.bfloat16),
            pltpu.VMEM((H_pad, N_pad), jnp.bfloat16),
            pltpu.VMEM((C_pad, N_pad), jnp.bfloat16),
            pltpu.SemaphoreType.DMA((N_pad // _SUB,)),
        ],
        compiler_params=pltpu.CompilerParams(
            dimension_semantics=("arbitrary", "arbitrary"),
            vmem_limit_bytes=60 << 20,
        ),
    )(x_p, w1_p, w2_p, b1_p, b2_p, adj_p)

    return out_p[:n_nodes, :num_classes]


def kernel(x_p, adj_p, w1_p, b1_p, w2_p, b2_p):
    return _gcn_forward(x_p, adj_p, w1_p, b1_p, w2_p, b2_p,
                        n_nodes=4096, num_classes=7, tm=1024)


# speedup vs baseline: 1.0769x; 1.0769x over previous
import functools

import jax
import jax.numpy as jnp
from jax.experimental import pallas as pl
from jax.experimental.pallas import tpu as pltpu

_RING = 2


def _fused_gcn_kernel(x_ref, w1_ref, w2_ref, b1_ref, b2_ref, adj_hbm,
                      out_ref, adjT_ref, ring_ref, s1t_ref, s2t_ref,
                      copy_sems, *, tm, num_classes, nr):
    phase = pl.program_id(0)
    i = pl.program_id(1)
    row0 = pl.multiple_of(i * tm, tm)

    def dma(j):
        return pltpu.make_async_copy(
            adj_hbm.at[pl.ds(j * tm, tm), :],
            ring_ref.at[jax.lax.rem(j, _RING)],
            copy_sems.at[j],
        )

    @pl.when(phase == 0)
    def _():
        @pl.when(i < _RING)
        def _():
            dma(i).start()

        p = jnp.dot(x_ref[...], w1_ref[...],
                    preferred_element_type=jnp.float32)
        s1t_ref[:, pl.ds(row0, tm)] = p.astype(jnp.bfloat16).T

    @pl.when(phase == 1)
    def _():
        dma(i).wait()
        adjT_ref[:, pl.ds(row0, tm)] = ring_ref[jax.lax.rem(i, _RING)].T

        @pl.when(i + _RING < nr)
        def _():
            dma(i + _RING).start()

        ht = jnp.dot(s1t_ref[...], adjT_ref[:, pl.ds(row0, tm)],
                     preferred_element_type=jnp.float32)
        ht = jnp.maximum(ht + b1_ref[...].T, 0.0)
        s2t_ref[:, pl.ds(row0, tm)] = jax.lax.dot_general(
            w2_ref[...], ht.astype(jnp.bfloat16),
            (((0,), (0,)), ((), ())),
            preferred_element_type=jnp.float32
        ).astype(s2t_ref.dtype)

    @pl.when(phase == 2)
    def _():
        zt = jnp.dot(s2t_ref[...], adjT_ref[:, pl.ds(row0, tm)],
                     preferred_element_type=jnp.float32)
        z = zt.T + b2_ref[...]
        lane = jax.lax.broadcasted_iota(jnp.int32, z.shape, 1)
        z = jnp.where(lane < num_classes, z, jnp.float32(-1e30))
        m = jnp.max(z, axis=1, keepdims=True)
        shifted = z - m
        lse = jnp.log(jnp.sum(jnp.exp(shifted), axis=1, keepdims=True))
        out_ref[...] = (shifted - lse).astype(out_ref.dtype)


@functools.partial(jax.jit, static_argnames=("n_nodes", "num_classes", "tm"))
def _gcn_forward(x_p, adj_p, w1_p, b1_p, w2_p, b2_p, *, n_nodes, num_classes,
                 tm):
    N_pad, F_pad = x_p.shape
    H_pad = w1_p.shape[1]
    C_pad = w2_p.shape[1]
    nr = N_pad // tm

    out_p = pl.pallas_call(
        functools.partial(_fused_gcn_kernel, tm=tm, num_classes=num_classes,
                          nr=nr),
        out_shape=jax.ShapeDtypeStruct((N_pad, C_pad), jnp.float32),
        grid=(3, nr),
        in_specs=[
            pl.BlockSpec((tm, F_pad),
                         lambda p, i: (jnp.where(p == 0, i, nr - 1), 0)),
            pl.BlockSpec((F_pad, H_pad), lambda p, i: (0, 0)),
            pl.BlockSpec((H_pad, C_pad), lambda p, i: (0, 0)),
            pl.BlockSpec((1, H_pad), lambda p, i: (0, 0)),
            pl.BlockSpec((1, C_pad), lambda p, i: (0, 0)),
            pl.BlockSpec(memory_space=pl.ANY),
        ],
        out_specs=pl.BlockSpec((tm, C_pad),
                               lambda p, i: (jnp.where(p == 2, i, 0), 0)),
        scratch_shapes=[
            pltpu.VMEM((N_pad, N_pad), jnp.bfloat16),
            pltpu.VMEM((_RING, tm, N_pad), jnp.bfloat16),
            pltpu.VMEM((H_pad, N_pad), jnp.bfloat16),
            pltpu.VMEM((C_pad, N_pad), jnp.bfloat16),
            pltpu.SemaphoreType.DMA((nr,)),
        ],
        compiler_params=pltpu.CompilerParams(
            dimension_semantics=("arbitrary", "arbitrary"),
            vmem_limit_bytes=60 << 20,
        ),
    )(x_p, w1_p, w2_p, b1_p, b2_p, adj_p)

    return out_p[:n_nodes, :num_classes]


def kernel(x_p, adj_p, w1_p, b1_p, w2_p, b2_p):
    return _gcn_forward(x_p, adj_p, w1_p, b1_p, w2_p, b2_p,
                        n_nodes=4096, num_classes=7, tm=1024)
